# baseline (device time: 37844 ns/iter reference)
import jax
import jax.numpy as jnp
from jax import lax
from jax.experimental import pallas as pl
from jax.experimental.pallas import tpu as pltpu

N_CHUNKS = 16


def kernel(x):
    m, n = x.shape
    half_m = m // 2
    chunk_m = half_m // N_CHUNKS

    def body(x_ref, out_ref, x_vmem, comm_ref, red_ref,
             p1_send, p1_recv, p2_send, p2_recv, xin_sem, oout_sem):
        my_x = lax.axis_index("x")
        my_y = lax.axis_index("y")
        y_nbr = (my_x, 1 - my_y)
        x_nbr = (1 - my_x, my_y)
        base = my_x * half_m
        other = (1 - my_x) * half_m

        xin = []
        for c in range(N_CHUNKS):
            cp = pltpu.make_async_copy(
                x_ref.at[pl.ds(base + c * chunk_m, chunk_m), :],
                x_vmem.at[c],
                xin_sem.at[c],
            )
            cp.start()
            xin.append(cp)

        barrier_sem = pltpu.get_barrier_semaphore()
        for nbr in (y_nbr, x_nbr):
            pl.semaphore_signal(
                barrier_sem, inc=1, device_id=nbr,
                device_id_type=pl.DeviceIdType.MESH,
            )
        pl.semaphore_wait(barrier_sem, 2)

        p1 = []
        for c in range(N_CHUNKS):
            rdma = pltpu.make_async_remote_copy(
                src_ref=x_ref.at[pl.ds(base + c * chunk_m, chunk_m), :],
                dst_ref=comm_ref.at[c],
                send_sem=p1_send.at[c],
                recv_sem=p1_recv.at[c],
                device_id=y_nbr,
                device_id_type=pl.DeviceIdType.MESH,
            )
            rdma.start()
            p1.append(rdma)

        p2, oout = [], []
        for c in range(N_CHUNKS):
            p1[c].wait_recv()
            xin[c].wait()
            rows = pl.ds(base + c * chunk_m, chunk_m)
            red_ref[c] = x_vmem[c] + comm_ref[c]
            cp = pltpu.make_async_copy(
                red_ref.at[c], out_ref.at[rows, :], oout_sem.at[c],
            )
            cp.start()
            oout.append(cp)
            rdma = pltpu.make_async_remote_copy(
                src_ref=red_ref.at[c],
                dst_ref=out_ref.at[rows, :],
                send_sem=p2_send.at[c],
                recv_sem=p2_recv.at[c],
                device_id=x_nbr,
                device_id_type=pl.DeviceIdType.MESH,
            )
            rdma.start()
            p2.append(rdma)

        for c in range(N_CHUNKS):
            recv = pltpu.make_async_remote_copy(
                src_ref=red_ref.at[c],
                dst_ref=out_ref.at[pl.ds(other + c * chunk_m, chunk_m), :],
                send_sem=p2_send.at[c],
                recv_sem=p2_recv.at[c],
                device_id=x_nbr,
                device_id_type=pl.DeviceIdType.MESH,
            )
            recv.wait_recv()
        for c in range(N_CHUNKS):
            p1[c].wait_send()
            p2[c].wait_send()
            oout[c].wait()

    return pl.pallas_call(
        body,
        out_shape=jax.ShapeDtypeStruct((m, n), x.dtype),
        in_specs=[pl.BlockSpec(memory_space=pl.ANY)],
        out_specs=pl.BlockSpec(memory_space=pl.ANY),
        scratch_shapes=[
            pltpu.VMEM((N_CHUNKS, chunk_m, n), x.dtype),
            pltpu.VMEM((N_CHUNKS, chunk_m, n), x.dtype),
            pltpu.VMEM((N_CHUNKS, chunk_m, n), x.dtype),
            pltpu.SemaphoreType.DMA((N_CHUNKS,)),
            pltpu.SemaphoreType.DMA((N_CHUNKS,)),
            pltpu.SemaphoreType.DMA((N_CHUNKS,)),
            pltpu.SemaphoreType.DMA((N_CHUNKS,)),
            pltpu.SemaphoreType.DMA((N_CHUNKS,)),
            pltpu.SemaphoreType.DMA((N_CHUNKS,)),
        ],
        compiler_params=pltpu.CompilerParams(collective_id=0),
    )(x)


# device time: 35015 ns/iter; 1.0808x vs baseline; 1.0808x over previous
import jax
import jax.numpy as jnp
from jax import lax
from jax.experimental import pallas as pl
from jax.experimental.pallas import tpu as pltpu

N_CHUNKS = 16


def kernel(x):
    m, n = x.shape
    half_m = m // 2
    chunk_m = half_m // N_CHUNKS

    def body(x_ref, out_ref, comm_ref, red_ref,
             p1_send, p1_recv, p2_send, p2_recv, oout_sem):
        my_x = lax.axis_index("x")
        my_y = lax.axis_index("y")
        y_nbr = (my_x, 1 - my_y)
        x_nbr = (1 - my_x, my_y)
        base = my_x * half_m
        other = (1 - my_x) * half_m

        barrier_sem = pltpu.get_barrier_semaphore()
        for nbr in (y_nbr, x_nbr):
            pl.semaphore_signal(
                barrier_sem, inc=1, device_id=nbr,
                device_id_type=pl.DeviceIdType.MESH,
            )
        pl.semaphore_wait(barrier_sem, 2)

        p1 = []
        for c in range(N_CHUNKS):
            rdma = pltpu.make_async_remote_copy(
                src_ref=x_ref.at[pl.ds(base + c * chunk_m, chunk_m), :],
                dst_ref=comm_ref.at[c],
                send_sem=p1_send.at[c],
                recv_sem=p1_recv.at[c],
                device_id=y_nbr,
                device_id_type=pl.DeviceIdType.MESH,
            )
            rdma.start()
            p1.append(rdma)

        p2, oout = [], []
        for c in range(N_CHUNKS):
            p1[c].wait_recv()
            rows = pl.ds(base + c * chunk_m, chunk_m)
            red_ref[c] = x_ref[rows, :] + comm_ref[c]
            cp = pltpu.make_async_copy(
                red_ref.at[c], out_ref.at[rows, :], oout_sem.at[c],
            )
            cp.start()
            oout.append(cp)
            rdma = pltpu.make_async_remote_copy(
                src_ref=red_ref.at[c],
                dst_ref=out_ref.at[rows, :],
                send_sem=p2_send.at[c],
                recv_sem=p2_recv.at[c],
                device_id=x_nbr,
                device_id_type=pl.DeviceIdType.MESH,
            )
            rdma.start()
            p2.append(rdma)

        for c in range(N_CHUNKS):
            recv = pltpu.make_async_remote_copy(
                src_ref=red_ref.at[c],
                dst_ref=out_ref.at[pl.ds(other + c * chunk_m, chunk_m), :],
                send_sem=p2_send.at[c],
                recv_sem=p2_recv.at[c],
                device_id=x_nbr,
                device_id_type=pl.DeviceIdType.MESH,
            )
            recv.wait_recv()
        for c in range(N_CHUNKS):
            p1[c].wait_send()
            p2[c].wait_send()
            oout[c].wait()

    return pl.pallas_call(
        body,
        out_shape=jax.ShapeDtypeStruct((m, n), x.dtype),
        in_specs=[pl.BlockSpec(memory_space=pltpu.VMEM)],
        out_specs=pl.BlockSpec(memory_space=pl.ANY),
        scratch_shapes=[
            pltpu.VMEM((N_CHUNKS, chunk_m, n), x.dtype),
            pltpu.VMEM((N_CHUNKS, chunk_m, n), x.dtype),
            pltpu.SemaphoreType.DMA((N_CHUNKS,)),
            pltpu.SemaphoreType.DMA((N_CHUNKS,)),
            pltpu.SemaphoreType.DMA((N_CHUNKS,)),
            pltpu.SemaphoreType.DMA((N_CHUNKS,)),
            pltpu.SemaphoreType.DMA((N_CHUNKS,)),
        ],
        compiler_params=pltpu.CompilerParams(collective_id=0),
    )(x)
